# baseline (speedup 1.0000x reference)
import functools
import math

import jax
import jax.numpy as jnp
from jax import lax
from jax.experimental import pallas as pl
from jax.experimental.pallas import tpu as pltpu
from jax.experimental.pallas import tpu_sc as plsc

_B, _S, _D, _H, _DH, _FF, _V, _P, _K, _L = 1, 2048, 1024, 16, 64, 4096, 50257, 64, 8, 2
_IMPORTANCE = 3.0


def _sc_gather(table, idx):
    info = plsc.get_sparse_core_info()
    nc, ns = info.num_cores, info.num_subcores
    nw = nc * ns
    n = idx.shape[0]
    d = table.shape[1]
    b_per_w = n // nw
    mesh = plsc.VectorSubcoreMesh(core_axis_name="c", subcore_axis_name="s")

    @functools.partial(
        pl.kernel,
        mesh=mesh,
        out_type=jax.ShapeDtypeStruct((n, d), table.dtype),
        scratch_types=[
            pltpu.VMEM((b_per_w,), jnp.int32),
            pltpu.VMEM((b_per_w, d), table.dtype),
            pltpu.SemaphoreType.DMA,
        ],
    )
    def k(table_hbm, idx_hbm, out_hbm, idx_v, rows_v, sem):
        wid = lax.axis_index("s") * nc + lax.axis_index("c")
        base = wid * b_per_w
        pltpu.sync_copy(idx_hbm.at[pl.ds(base, b_per_w)], idx_v)
        pltpu.async_copy(table_hbm.at[idx_v], rows_v, sem).wait()
        pltpu.sync_copy(rows_v, out_hbm.at[pl.ds(base, b_per_w)])

    return k(table, idx)


def _mm_bias(x, w, b, bm=512, bn=512):
    m, kd = x.shape
    n = w.shape[1]
    bm = min(bm, m)
    bn = min(bn, n)
    while m % bm:
        bm //= 2
    while n % bn:
        bn //= 2

    def body(x_ref, w_ref, b_ref, o_ref):
        o_ref[...] = (
            jnp.dot(x_ref[...], w_ref[...], preferred_element_type=jnp.float32)
            + b_ref[...]
        )

    return pl.pallas_call(
        body,
        grid=(m // bm, n // bn),
        in_specs=[
            pl.BlockSpec((bm, kd), lambda i, j: (i, 0)),
            pl.BlockSpec((kd, bn), lambda i, j: (0, j)),
            pl.BlockSpec((1, bn), lambda i, j: (0, j)),
        ],
        out_specs=pl.BlockSpec((bm, bn), lambda i, j: (i, j)),
        out_shape=jax.ShapeDtypeStruct((m, n), jnp.float32),
    )(x, w, b.reshape(1, n))


def _attention(q3, k3, v3):
    sq = q3.shape[1]
    s = k3.shape[1]
    bq = min(512, sq)
    scale = 1.0 / math.sqrt(_DH)

    def body(q_ref, k_ref, v_ref, o_ref):
        q = q_ref[0]
        k = k_ref[0]
        sc = lax.dot_general(
            q, k, (((1,), (1,)), ((), ())), preferred_element_type=jnp.float32
        ) * scale
        mx = jnp.max(sc, axis=1, keepdims=True)
        e = jnp.exp(sc - mx)
        p = e / jnp.sum(e, axis=1, keepdims=True)
        o_ref[0] = jnp.dot(p, v_ref[0], preferred_element_type=jnp.float32)

    return pl.pallas_call(
        body,
        grid=(_H, sq // bq),
        in_specs=[
            pl.BlockSpec((1, bq, _DH), lambda h, i: (h, i, 0)),
            pl.BlockSpec((1, s, _DH), lambda h, i: (h, 0, 0)),
            pl.BlockSpec((1, s, _DH), lambda h, i: (h, 0, 0)),
        ],
        out_specs=pl.BlockSpec((1, bq, _DH), lambda h, i: (h, i, 0)),
        out_shape=jax.ShapeDtypeStruct((_H, sq, _DH), jnp.float32),
    )(q3, k3, v3)


def _ln_body(h, g, b):
    mu = jnp.mean(h, axis=1, keepdims=True)
    var = jnp.mean((h - mu) * (h - mu), axis=1, keepdims=True)
    return (h - mu) * lax.rsqrt(var + 1e-5) * g + b


def _proj_ln(x, o, wo, bo, g, b):
    m = x.shape[0]
    bm = min(256, m)

    def body(x_ref, o_ref, wo_ref, bo_ref, g_ref, b_ref, out_ref):
        h = (
            x_ref[...]
            + jnp.dot(o_ref[...], wo_ref[...], preferred_element_type=jnp.float32)
            + bo_ref[...]
        )
        out_ref[...] = _ln_body(h, g_ref[...], b_ref[...])

    vec = lambda i: (0, 0)
    return pl.pallas_call(
        body,
        grid=(m // bm,),
        in_specs=[
            pl.BlockSpec((bm, _D), lambda i: (i, 0)),
            pl.BlockSpec((bm, _D), lambda i: (i, 0)),
            pl.BlockSpec((_D, _D), vec),
            pl.BlockSpec((1, _D), vec),
            pl.BlockSpec((1, _D), vec),
            pl.BlockSpec((1, _D), vec),
        ],
        out_specs=pl.BlockSpec((bm, _D), lambda i: (i, 0)),
        out_shape=jax.ShapeDtypeStruct((m, _D), jnp.float32),
    )(x, o, wo, bo.reshape(1, _D), g.reshape(1, _D), b.reshape(1, _D))


def _ffn_ln(x, w1, b1, w2, b2, g, b):
    m = x.shape[0]
    bm = min(256, m)

    def body(x_ref, w1_ref, b1_ref, w2_ref, b2_ref, g_ref, b_ref, out_ref):
        xv = x_ref[...]
        mid = jnp.maximum(
            jnp.dot(xv, w1_ref[...], preferred_element_type=jnp.float32)
            + b1_ref[...],
            0.0,
        )
        h = (
            xv
            + jnp.dot(mid, w2_ref[...], preferred_element_type=jnp.float32)
            + b2_ref[...]
        )
        out_ref[...] = _ln_body(h, g_ref[...], b_ref[...])

    vec = lambda i: (0, 0)
    return pl.pallas_call(
        body,
        grid=(m // bm,),
        in_specs=[
            pl.BlockSpec((bm, _D), lambda i: (i, 0)),
            pl.BlockSpec((_D, _FF), vec),
            pl.BlockSpec((1, _FF), vec),
            pl.BlockSpec((_FF, _D), vec),
            pl.BlockSpec((1, _D), vec),
            pl.BlockSpec((1, _D), vec),
            pl.BlockSpec((1, _D), vec),
        ],
        out_specs=pl.BlockSpec((bm, _D), lambda i: (i, 0)),
        out_shape=jax.ShapeDtypeStruct((m, _D), jnp.float32),
    )(x, w1, b1.reshape(1, _FF), w2, b2.reshape(1, _D), g.reshape(1, _D),
      b.reshape(1, _D))


def _split_heads(t):
    return t.reshape(t.shape[0], _H, _DH).transpose(1, 0, 2)


def _encoder_layer(x, wqkv, bqkv, wo, bo, w1, b1, w2, b2, g1, be1, g2, be2):
    qkv = _mm_bias(x, wqkv, bqkv)
    q3 = _split_heads(qkv[:, :_D])
    k3 = _split_heads(qkv[:, _D:2 * _D])
    v3 = _split_heads(qkv[:, 2 * _D:])
    o = _attention(q3, k3, v3).transpose(1, 0, 2).reshape(-1, _D)
    x = _proj_ln(x, o, wo, bo, g1, be1)
    return _ffn_ln(x, w1, b1, w2, b2, g2, be2)


def _route(lc_last, gate_w, gate_b):
    scale = math.sqrt(_D)

    def body(x_ref, w_ref, b_ref, jw_ref, idx_ref):
        logits = scale * lax.dot_general(
            x_ref[...], w_ref[...], (((1,), (1,)), ((), ())),
            preferred_element_type=jnp.float32,
        ) + b_ref[...]
        iota = lax.broadcasted_iota(jnp.int32, (1, _P), 1)
        vals = logits
        tv, ti = [], []
        for _ in range(_K):
            mv = jnp.max(vals)
            ix = jnp.min(jnp.where(vals == mv, iota, _P))
            tv.append(mv)
            ti.append(ix)
            vals = jnp.where(iota == ix, -1e30, vals)
        m0 = tv[0]
        evec = jnp.exp(logits - m0)
        tot = jnp.float32(0.0)
        num = []
        for i in range(_K):
            ei = jnp.sum(jnp.where(iota == ti[i], evec, 0.0))
            num.append(ei)
            tot = tot + ei
        for i in range(_K):
            jw_ref[i] = num[i] / tot
            idx_ref[i] = ti[i]

    return pl.pallas_call(
        body,
        in_specs=[
            pl.BlockSpec((1, _D), lambda: (0, 0)),
            pl.BlockSpec((_P, _D), lambda: (0, 0)),
            pl.BlockSpec((1, _P), lambda: (0, 0)),
        ],
        out_specs=[
            pl.BlockSpec(memory_space=pltpu.SMEM),
            pl.BlockSpec(memory_space=pltpu.SMEM),
        ],
        out_shape=[
            jax.ShapeDtypeStruct((_K,), jnp.float32),
            jax.ShapeDtypeStruct((_K,), jnp.int32),
        ],
    )(lc_last, gate_w, gate_b.reshape(1, _P))


def _combine(jw, resp):
    bm = 256

    def body(jw_ref, r_ref, o_ref):
        acc = jw_ref[0] * r_ref[0]
        for kk in range(1, _K):
            acc = acc + jw_ref[kk] * r_ref[kk]
        o_ref[...] = acc

    return pl.pallas_call(
        body,
        grid=(_S // bm,),
        in_specs=[
            pl.BlockSpec(memory_space=pltpu.SMEM),
            pl.BlockSpec((_K, bm, _D), lambda i: (0, i, 0)),
        ],
        out_specs=pl.BlockSpec((bm, _D), lambda i: (i, 0)),
        out_shape=jax.ShapeDtypeStruct((_S, _D), jnp.float32),
    )(jw, resp)


_BV = 512
_NV = (_V + _BV - 1) // _BV


def _decode_lse(h, dec_w):
    def body(h_ref, w_ref, out_ref, lse_ref, m_s, s_s):
        v = pl.program_id(0)

        @pl.when(v == 0)
        def _():
            m_s[...] = jnp.full_like(m_s, -1e30)
            s_s[...] = jnp.zeros_like(s_s)

        logits = lax.dot_general(
            h_ref[...], w_ref[...], (((1,), (1,)), ((), ())),
            preferred_element_type=jnp.float32,
        )
        out_ref[0] = logits
        col = lax.broadcasted_iota(jnp.int32, (_S, _BV), 1) + v * _BV
        lm = jnp.where(col < _V, logits, -1e30)
        bmax = jnp.max(lm, axis=1, keepdims=True)
        esum = jnp.sum(jnp.exp(lm - bmax), axis=1, keepdims=True)
        m_old = m_s[...]
        m_new = jnp.maximum(m_old, bmax)
        s_new = s_s[...] * jnp.exp(m_old - m_new) + jnp.exp(bmax - m_new) * esum
        m_s[...] = m_new
        s_s[...] = s_new

        @pl.when(v == _NV - 1)
        def _():
            lse_ref[...] = m_new + jnp.log(s_new)

    return pl.pallas_call(
        body,
        grid=(_NV,),
        in_specs=[
            pl.BlockSpec((_S, _D), lambda v: (0, 0)),
            pl.BlockSpec((_BV, _D), lambda v: (v, 0)),
        ],
        out_specs=[
            pl.BlockSpec((1, _S, _BV), lambda v: (0, 0, v)),
            pl.BlockSpec((_S, 128), lambda v: (0, 0)),
        ],
        out_shape=[
            jax.ShapeDtypeStruct((1, _S, _V), jnp.float32),
            jax.ShapeDtypeStruct((_S, 128), jnp.float32),
        ],
        scratch_shapes=[
            pltpu.VMEM((_S, 128), jnp.float32),
            pltpu.VMEM((_S, 128), jnp.float32),
        ],
    )(h, dec_w)


def _finalize(h, labrows, lse, tw):
    def body(h_ref, lab_ref, lse_ref, tw_ref, out_ref):
        dot = jnp.sum(h_ref[...] * lab_ref[...], axis=1, keepdims=True)
        lse_val = jnp.max(lse_ref[...], axis=1, keepdims=True)
        row = lax.broadcasted_iota(jnp.int32, (_S, 1), 0)
        ll = jnp.where(row < _S - 1, dot - lse_val, 0.0)
        ce = -jnp.sum(ll) / (_S - 1)
        twv = tw_ref[...]
        mu = jnp.sum(twv) / _P
        var = jnp.sum((twv - mu) * (twv - mu)) / (_P - 1)
        out_ref[0, 0] = ce + _IMPORTANCE * var / (mu * mu)

    return pl.pallas_call(
        body,
        in_specs=[
            pl.BlockSpec((_S, _D), lambda: (0, 0)),
            pl.BlockSpec((_S, _D), lambda: (0, 0)),
            pl.BlockSpec((_S, 128), lambda: (0, 0)),
            pl.BlockSpec((1, _P), lambda: (0, 0)),
        ],
        out_specs=pl.BlockSpec(memory_space=pltpu.SMEM),
        out_shape=jax.ShapeDtypeStruct((1, 1), jnp.float32),
    )(h, labrows, lse, tw)


def kernel(inputs, responses, emb_table, dec_w, gate_w, gate_b, total_weights,
           le_wqkv, le_bqkv, le_wo, le_bo, le_w1, le_b1, le_w2, le_b2,
           le_ln1g, le_ln1b, le_ln2g, le_ln2b,
           enc_wqkv, enc_bqkv, enc_wo, enc_bo, enc_w1, enc_b1, enc_w2, enc_b2,
           enc_ln1g, enc_ln1b, enc_ln2g, enc_ln2b):
    idx = inputs[0].astype(jnp.int32)

    emb = _sc_gather(emb_table, idx)
    kv = _mm_bias(emb, le_wqkv[:, _D:], le_bqkv[_D:])
    q_last = _mm_bias(emb[_S - 1:], le_wqkv[:, :_D], le_bqkv[:_D])
    k3 = _split_heads(kv[:, :_D])
    v3 = _split_heads(kv[:, _D:])
    q3 = _split_heads(q_last)
    o = _attention(q3, k3, v3).transpose(1, 0, 2).reshape(1, _D)
    x1 = _proj_ln(emb[_S - 1:], o, le_wo, le_bo, le_ln1g, le_ln1b)
    lc_last = _ffn_ln(x1, le_w1, le_b1, le_w2, le_b2, le_ln2g, le_ln2b)

    jw, topk_idx = _route(lc_last, gate_w, gate_b)
    h = _combine(jw, responses.reshape(_K, _S, _D))

    for l in range(_L):
        h = _encoder_layer(
            h, enc_wqkv[l], enc_bqkv[l], enc_wo[l], enc_bo[l],
            enc_w1[l], enc_b1[l], enc_w2[l], enc_b2[l],
            enc_ln1g[l], enc_ln1b[l], enc_ln2g[l], enc_ln2b[l])

    decoded, lse = _decode_lse(h, dec_w)
    lab = jnp.concatenate([idx[1:], jnp.zeros((1,), jnp.int32)])
    labrows = _sc_gather(dec_w, lab)
    loss = _finalize(h, labrows, lse, total_weights.reshape(1, _P))

    return (loss.reshape(()), decoded, topk_idx)


# speedup vs baseline: 1.3296x; 1.3296x over previous
import functools
import math

import jax
import jax.numpy as jnp
from jax import lax
from jax.experimental import pallas as pl
from jax.experimental.pallas import tpu as pltpu
from jax.experimental.pallas import tpu_sc as plsc

_B, _S, _D, _H, _DH, _FF, _V, _P, _K, _L = 1, 2048, 1024, 16, 64, 4096, 50257, 64, 8, 2
_IMPORTANCE = 3.0


def _sc_gather(table, idx):
    info = plsc.get_sparse_core_info()
    nc, ns = info.num_cores, info.num_subcores
    nw = nc * ns
    n = idx.shape[0]
    d = table.shape[1]
    b_per_w = n // nw
    mesh = plsc.VectorSubcoreMesh(core_axis_name="c", subcore_axis_name="s")

    @functools.partial(
        pl.kernel,
        mesh=mesh,
        out_type=jax.ShapeDtypeStruct((n, d), table.dtype),
        scratch_types=[
            pltpu.VMEM((b_per_w,), jnp.int32),
            pltpu.VMEM((b_per_w, d), table.dtype),
            pltpu.SemaphoreType.DMA,
        ],
    )
    def k(table_hbm, idx_hbm, out_hbm, idx_v, rows_v, sem):
        wid = lax.axis_index("s") * nc + lax.axis_index("c")
        base = wid * b_per_w
        pltpu.sync_copy(idx_hbm.at[pl.ds(base, b_per_w)], idx_v)
        pltpu.async_copy(table_hbm.at[idx_v], rows_v, sem).wait()
        pltpu.sync_copy(rows_v, out_hbm.at[pl.ds(base, b_per_w)])

    return k(table, idx)


def _mm_bias(x, w, b, bm=512, bn=512):
    m, kd = x.shape
    n = w.shape[1]
    bm = min(bm, m)
    bn = min(bn, n)
    while m % bm:
        bm //= 2
    while n % bn:
        bn //= 2

    def body(x_ref, w_ref, b_ref, o_ref):
        o_ref[...] = (
            jnp.dot(x_ref[...], w_ref[...], preferred_element_type=jnp.float32)
            + b_ref[...]
        )

    return pl.pallas_call(
        body,
        grid=(m // bm, n // bn),
        in_specs=[
            pl.BlockSpec((bm, kd), lambda i, j: (i, 0)),
            pl.BlockSpec((kd, bn), lambda i, j: (0, j)),
            pl.BlockSpec((1, bn), lambda i, j: (0, j)),
        ],
        out_specs=pl.BlockSpec((bm, bn), lambda i, j: (i, j)),
        out_shape=jax.ShapeDtypeStruct((m, n), jnp.float32),
    )(x, w, b.reshape(1, n))


def _attention(q3, k3, v3):
    sq = q3.shape[1]
    s = k3.shape[1]
    bq = min(512, sq)
    scale = 1.0 / math.sqrt(_DH)

    def body(q_ref, k_ref, v_ref, o_ref):
        q = q_ref[0]
        k = k_ref[0]
        sc = lax.dot_general(
            q, k, (((1,), (1,)), ((), ())), preferred_element_type=jnp.float32
        ) * scale
        mx = jnp.max(sc, axis=1, keepdims=True)
        e = jnp.exp(sc - mx)
        p = e / jnp.sum(e, axis=1, keepdims=True)
        o_ref[0] = jnp.dot(p, v_ref[0], preferred_element_type=jnp.float32)

    return pl.pallas_call(
        body,
        grid=(_H, sq // bq),
        in_specs=[
            pl.BlockSpec((1, bq, _DH), lambda h, i: (h, i, 0)),
            pl.BlockSpec((1, s, _DH), lambda h, i: (h, 0, 0)),
            pl.BlockSpec((1, s, _DH), lambda h, i: (h, 0, 0)),
        ],
        out_specs=pl.BlockSpec((1, bq, _DH), lambda h, i: (h, i, 0)),
        out_shape=jax.ShapeDtypeStruct((_H, sq, _DH), jnp.float32),
    )(q3, k3, v3)


def _ln_body(h, g, b):
    mu = jnp.mean(h, axis=1, keepdims=True)
    var = jnp.mean((h - mu) * (h - mu), axis=1, keepdims=True)
    return (h - mu) * lax.rsqrt(var + 1e-5) * g + b


def _proj_ln(x, o, wo, bo, g, b):
    m = x.shape[0]
    bm = min(256, m)

    def body(x_ref, o_ref, wo_ref, bo_ref, g_ref, b_ref, out_ref):
        h = (
            x_ref[...]
            + jnp.dot(o_ref[...], wo_ref[...], preferred_element_type=jnp.float32)
            + bo_ref[...]
        )
        out_ref[...] = _ln_body(h, g_ref[...], b_ref[...])

    vec = lambda i: (0, 0)
    return pl.pallas_call(
        body,
        grid=(m // bm,),
        in_specs=[
            pl.BlockSpec((bm, _D), lambda i: (i, 0)),
            pl.BlockSpec((bm, _D), lambda i: (i, 0)),
            pl.BlockSpec((_D, _D), vec),
            pl.BlockSpec((1, _D), vec),
            pl.BlockSpec((1, _D), vec),
            pl.BlockSpec((1, _D), vec),
        ],
        out_specs=pl.BlockSpec((bm, _D), lambda i: (i, 0)),
        out_shape=jax.ShapeDtypeStruct((m, _D), jnp.float32),
    )(x, o, wo, bo.reshape(1, _D), g.reshape(1, _D), b.reshape(1, _D))


def _ffn_ln(x, w1, b1, w2, b2, g, b):
    m = x.shape[0]
    bm = min(256, m)

    def body(x_ref, w1_ref, b1_ref, w2_ref, b2_ref, g_ref, b_ref, out_ref):
        xv = x_ref[...]
        mid = jnp.maximum(
            jnp.dot(xv, w1_ref[...], preferred_element_type=jnp.float32)
            + b1_ref[...],
            0.0,
        )
        h = (
            xv
            + jnp.dot(mid, w2_ref[...], preferred_element_type=jnp.float32)
            + b2_ref[...]
        )
        out_ref[...] = _ln_body(h, g_ref[...], b_ref[...])

    vec = lambda i: (0, 0)
    return pl.pallas_call(
        body,
        grid=(m // bm,),
        in_specs=[
            pl.BlockSpec((bm, _D), lambda i: (i, 0)),
            pl.BlockSpec((_D, _FF), vec),
            pl.BlockSpec((1, _FF), vec),
            pl.BlockSpec((_FF, _D), vec),
            pl.BlockSpec((1, _D), vec),
            pl.BlockSpec((1, _D), vec),
            pl.BlockSpec((1, _D), vec),
        ],
        out_specs=pl.BlockSpec((bm, _D), lambda i: (i, 0)),
        out_shape=jax.ShapeDtypeStruct((m, _D), jnp.float32),
    )(x, w1, b1.reshape(1, _FF), w2, b2.reshape(1, _D), g.reshape(1, _D),
      b.reshape(1, _D))


def _split_heads(t):
    return t.reshape(t.shape[0], _H, _DH).transpose(1, 0, 2)


def _encoder_layer(x, wqkv, bqkv, wo, bo, w1, b1, w2, b2, g1, be1, g2, be2):
    qkv = _mm_bias(x, wqkv, bqkv)
    q3 = _split_heads(qkv[:, :_D])
    k3 = _split_heads(qkv[:, _D:2 * _D])
    v3 = _split_heads(qkv[:, 2 * _D:])
    o = _attention(q3, k3, v3).transpose(1, 0, 2).reshape(-1, _D)
    x = _proj_ln(x, o, wo, bo, g1, be1)
    return _ffn_ln(x, w1, b1, w2, b2, g2, be2)


def _route(lc_last, gate_w, gate_b):
    scale = math.sqrt(_D)

    def body(x_ref, w_ref, b_ref, jw_ref, idx_ref):
        logits = scale * lax.dot_general(
            x_ref[...], w_ref[...], (((1,), (1,)), ((), ())),
            preferred_element_type=jnp.float32,
        ) + b_ref[...]
        iota = lax.broadcasted_iota(jnp.int32, (1, _P), 1)
        vals = logits
        tv, ti = [], []
        for _ in range(_K):
            mv = jnp.max(vals)
            ix = jnp.min(jnp.where(vals == mv, iota, _P))
            tv.append(mv)
            ti.append(ix)
            vals = jnp.where(iota == ix, -1e30, vals)
        m0 = tv[0]
        evec = jnp.exp(logits - m0)
        tot = jnp.float32(0.0)
        num = []
        for i in range(_K):
            ei = jnp.sum(jnp.where(iota == ti[i], evec, 0.0))
            num.append(ei)
            tot = tot + ei
        for i in range(_K):
            jw_ref[i] = num[i] / tot
            idx_ref[i] = ti[i]

    return pl.pallas_call(
        body,
        in_specs=[
            pl.BlockSpec((1, _D), lambda: (0, 0)),
            pl.BlockSpec((_P, _D), lambda: (0, 0)),
            pl.BlockSpec((1, _P), lambda: (0, 0)),
        ],
        out_specs=[
            pl.BlockSpec(memory_space=pltpu.SMEM),
            pl.BlockSpec(memory_space=pltpu.SMEM),
        ],
        out_shape=[
            jax.ShapeDtypeStruct((_K,), jnp.float32),
            jax.ShapeDtypeStruct((_K,), jnp.int32),
        ],
    )(lc_last, gate_w, gate_b.reshape(1, _P))


def _combine(jw, resp):
    bm = 256

    def body(jw_ref, r_ref, o_ref):
        acc = jw_ref[0] * r_ref[0]
        for kk in range(1, _K):
            acc = acc + jw_ref[kk] * r_ref[kk]
        o_ref[...] = acc

    return pl.pallas_call(
        body,
        grid=(_S // bm,),
        in_specs=[
            pl.BlockSpec(memory_space=pltpu.SMEM),
            pl.BlockSpec((_K, bm, _D), lambda i: (0, i, 0)),
        ],
        out_specs=pl.BlockSpec((bm, _D), lambda i: (i, 0)),
        out_shape=jax.ShapeDtypeStruct((_S, _D), jnp.float32),
    )(jw, resp)


_BV = 512
_NV = (_V + _BV - 1) // _BV


def _decode_lse(h, dec_w, lab):
    def body(h_ref, w_ref, lab_ref, out_ref, ll_ref, m_s, s_s, la_s):
        v = pl.program_id(0)

        @pl.when(v == 0)
        def _():
            m_s[...] = jnp.full_like(m_s, -1e30)
            s_s[...] = jnp.zeros_like(s_s)
            la_s[...] = jnp.zeros_like(la_s)

        logits = lax.dot_general(
            h_ref[...], w_ref[...], (((1,), (1,)), ((), ())),
            preferred_element_type=jnp.float32,
        )
        out_ref[...] = logits
        col = lax.broadcasted_iota(jnp.int32, (_S, _BV), 1) + v * _BV
        lm = jnp.where(col < _V, logits, -1e30)
        lab_col = lab_ref[...]
        la_s[...] = la_s[...] + jnp.sum(
            jnp.where(col == lab_col, logits, 0.0), axis=1, keepdims=True)
        bmax = jnp.max(lm, axis=1, keepdims=True)
        esum = jnp.sum(jnp.exp(lm - bmax), axis=1, keepdims=True)
        m_old = m_s[...]
        m_new = jnp.maximum(m_old, bmax)
        s_new = s_s[...] * jnp.exp(m_old - m_new) + jnp.exp(bmax - m_new) * esum
        m_s[...] = m_new
        s_s[...] = s_new

        @pl.when(v == _NV - 1)
        def _():
            ll_ref[...] = la_s[...] - (m_new + jnp.log(s_new))

    return pl.pallas_call(
        body,
        grid=(_NV,),
        in_specs=[
            pl.BlockSpec((_S, _D), lambda v: (0, 0)),
            pl.BlockSpec((_BV, _D), lambda v: (v, 0)),
            pl.BlockSpec((_S, 1), lambda v: (0, 0)),
        ],
        out_specs=[
            pl.BlockSpec((_S, _BV), lambda v: (0, v)),
            pl.BlockSpec((_S, 8), lambda v: (0, 0)),
        ],
        out_shape=[
            jax.ShapeDtypeStruct((_S, _V), jnp.float32),
            jax.ShapeDtypeStruct((_S, 8), jnp.float32),
        ],
        scratch_shapes=[
            pltpu.VMEM((_S, 8), jnp.float32),
            pltpu.VMEM((_S, 8), jnp.float32),
            pltpu.VMEM((_S, 8), jnp.float32),
        ],
    )(h, dec_w, lab.reshape(_S, 1))


def _finalize(ll, tw):
    def body(ll_ref, tw_ref, out_ref):
        llv = jnp.max(ll_ref[...], axis=1, keepdims=True)
        row = lax.broadcasted_iota(jnp.int32, (_S, 1), 0)
        ce = -jnp.sum(jnp.where(row < _S - 1, llv, 0.0)) / (_S - 1)
        twv = tw_ref[...]
        mu = jnp.sum(twv) / _P
        var = jnp.sum((twv - mu) * (twv - mu)) / (_P - 1)
        out_ref[0, 0] = ce + _IMPORTANCE * var / (mu * mu)

    return pl.pallas_call(
        body,
        in_specs=[
            pl.BlockSpec((_S, 8), lambda: (0, 0)),
            pl.BlockSpec((1, _P), lambda: (0, 0)),
        ],
        out_specs=pl.BlockSpec(memory_space=pltpu.SMEM),
        out_shape=jax.ShapeDtypeStruct((1, 1), jnp.float32),
    )(ll, tw)


def kernel(inputs, responses, emb_table, dec_w, gate_w, gate_b, total_weights,
           le_wqkv, le_bqkv, le_wo, le_bo, le_w1, le_b1, le_w2, le_b2,
           le_ln1g, le_ln1b, le_ln2g, le_ln2b,
           enc_wqkv, enc_bqkv, enc_wo, enc_bo, enc_w1, enc_b1, enc_w2, enc_b2,
           enc_ln1g, enc_ln1b, enc_ln2g, enc_ln2b):
    idx = inputs[0].astype(jnp.int32)

    emb = _sc_gather(emb_table, idx)
    kv = _mm_bias(emb, le_wqkv[:, _D:], le_bqkv[_D:])
    q_last = _mm_bias(emb[_S - 1:], le_wqkv[:, :_D], le_bqkv[:_D])
    k3 = _split_heads(kv[:, :_D])
    v3 = _split_heads(kv[:, _D:])
    q3 = _split_heads(q_last)
    o = _attention(q3, k3, v3).transpose(1, 0, 2).reshape(1, _D)
    x1 = _proj_ln(emb[_S - 1:], o, le_wo, le_bo, le_ln1g, le_ln1b)
    lc_last = _ffn_ln(x1, le_w1, le_b1, le_w2, le_b2, le_ln2g, le_ln2b)

    jw, topk_idx = _route(lc_last, gate_w, gate_b)
    h = _combine(jw, responses.reshape(_K, _S, _D))

    for l in range(_L):
        h = _encoder_layer(
            h, enc_wqkv[l], enc_bqkv[l], enc_wo[l], enc_bo[l],
            enc_w1[l], enc_b1[l], enc_w2[l], enc_b2[l],
            enc_ln1g[l], enc_ln1b[l], enc_ln2g[l], enc_ln2b[l])

    lab = jnp.concatenate([idx[1:], jnp.zeros((1,), jnp.int32)])
    decoded, ll = _decode_lse(h, dec_w, lab)
    loss = _finalize(ll, total_weights.reshape(1, _P))

    return (loss.reshape(()), decoded.reshape(1, _S, _V), topk_idx)


# speedup vs baseline: 1.6364x; 1.2307x over previous
import functools
import math

import jax
import jax.numpy as jnp
from jax import lax
from jax.experimental import pallas as pl
from jax.experimental.pallas import tpu as pltpu
from jax.experimental.pallas import tpu_sc as plsc

_B, _S, _D, _H, _DH, _FF, _V, _P, _K, _L = 1, 2048, 1024, 16, 64, 4096, 50257, 64, 8, 2
_IMPORTANCE = 3.0


def _sc_gather(table, idx):
    info = plsc.get_sparse_core_info()
    nc, ns = info.num_cores, info.num_subcores
    nw = nc * ns
    n = idx.shape[0]
    d = table.shape[1]
    b_per_w = n // nw
    mesh = plsc.VectorSubcoreMesh(core_axis_name="c", subcore_axis_name="s")

    @functools.partial(
        pl.kernel,
        mesh=mesh,
        out_type=jax.ShapeDtypeStruct((n, d), table.dtype),
        scratch_types=[
            pltpu.VMEM((b_per_w,), jnp.int32),
            pltpu.VMEM((b_per_w, d), table.dtype),
            pltpu.SemaphoreType.DMA,
        ],
    )
    def k(table_hbm, idx_hbm, out_hbm, idx_v, rows_v, sem):
        wid = lax.axis_index("s") * nc + lax.axis_index("c")
        base = wid * b_per_w
        pltpu.sync_copy(idx_hbm.at[pl.ds(base, b_per_w)], idx_v)
        pltpu.async_copy(table_hbm.at[idx_v], rows_v, sem).wait()
        pltpu.sync_copy(rows_v, out_hbm.at[pl.ds(base, b_per_w)])

    return k(table, idx)


def _mm_bias(x, w, b, bm=512, bn=512):
    m, kd = x.shape
    n = w.shape[1]
    bm = min(bm, m)
    bn = min(bn, n)
    while m % bm:
        bm //= 2
    while n % bn:
        bn //= 2

    def body(x_ref, w_ref, b_ref, o_ref):
        o_ref[...] = (
            jnp.dot(x_ref[...], w_ref[...], preferred_element_type=jnp.float32)
            + b_ref[...]
        )

    return pl.pallas_call(
        body,
        grid=(m // bm, n // bn),
        in_specs=[
            pl.BlockSpec((bm, kd), lambda i, j: (i, 0)),
            pl.BlockSpec((kd, bn), lambda i, j: (0, j)),
            pl.BlockSpec((1, bn), lambda i, j: (0, j)),
        ],
        out_specs=pl.BlockSpec((bm, bn), lambda i, j: (i, j)),
        out_shape=jax.ShapeDtypeStruct((m, n), jnp.float32),
    )(x, w, b.reshape(1, n))


def _attention(qarr, kvarr, q_off, k_off, v_off):
    sq = qarr.shape[0]
    s = kvarr.shape[0]
    bq = min(512, sq)
    scale = 1.0 / math.sqrt(_DH)

    def body(q_ref, k_ref, v_ref, o_ref):
        q = q_ref[...]
        k = k_ref[...]
        v = v_ref[...]
        outs = []
        for t in (0, 1):
            qt = q[:, t * _DH:(t + 1) * _DH]
            kt = k[:, t * _DH:(t + 1) * _DH]
            vt = v[:, t * _DH:(t + 1) * _DH]
            sc = lax.dot_general(
                qt, kt, (((1,), (1,)), ((), ())),
                preferred_element_type=jnp.float32) * scale
            mx = jnp.max(sc, axis=1, keepdims=True)
            e = jnp.exp(sc - mx)
            p = e / jnp.sum(e, axis=1, keepdims=True)
            outs.append(jnp.dot(p, vt, preferred_element_type=jnp.float32))
        o_ref[...] = jnp.concatenate(outs, axis=1)

    return pl.pallas_call(
        body,
        grid=(_H // 2, sq // bq),
        in_specs=[
            pl.BlockSpec((bq, 128), lambda h, i: (i, q_off + h)),
            pl.BlockSpec((s, 128), lambda h, i: (0, k_off + h)),
            pl.BlockSpec((s, 128), lambda h, i: (0, v_off + h)),
        ],
        out_specs=pl.BlockSpec((bq, 128), lambda h, i: (i, h)),
        out_shape=jax.ShapeDtypeStruct((sq, _D), jnp.float32),
    )(qarr, kvarr, kvarr)


def _ln_body(h, g, b):
    mu = jnp.mean(h, axis=1, keepdims=True)
    var = jnp.mean((h - mu) * (h - mu), axis=1, keepdims=True)
    return (h - mu) * lax.rsqrt(var + 1e-5) * g + b


def _proj_ln(x, o, wo, bo, g, b):
    m = x.shape[0]
    bm = min(256, m)

    def body(x_ref, o_ref, wo_ref, bo_ref, g_ref, b_ref, out_ref):
        h = (
            x_ref[...]
            + jnp.dot(o_ref[...], wo_ref[...], preferred_element_type=jnp.float32)
            + bo_ref[...]
        )
        out_ref[...] = _ln_body(h, g_ref[...], b_ref[...])

    vec = lambda i: (0, 0)
    return pl.pallas_call(
        body,
        grid=(m // bm,),
        in_specs=[
            pl.BlockSpec((bm, _D), lambda i: (i, 0)),
            pl.BlockSpec((bm, _D), lambda i: (i, 0)),
            pl.BlockSpec((_D, _D), vec),
            pl.BlockSpec((1, _D), vec),
            pl.BlockSpec((1, _D), vec),
            pl.BlockSpec((1, _D), vec),
        ],
        out_specs=pl.BlockSpec((bm, _D), lambda i: (i, 0)),
        out_shape=jax.ShapeDtypeStruct((m, _D), jnp.float32),
    )(x, o, wo, bo.reshape(1, _D), g.reshape(1, _D), b.reshape(1, _D))


def _ffn_ln(x, w1, b1, w2, b2, g, b):
    m = x.shape[0]
    bm = min(256, m)

    def body(x_ref, w1_ref, b1_ref, w2_ref, b2_ref, g_ref, b_ref, out_ref):
        xv = x_ref[...]
        mid = jnp.maximum(
            jnp.dot(xv, w1_ref[...], preferred_element_type=jnp.float32)
            + b1_ref[...],
            0.0,
        )
        h = (
            xv
            + jnp.dot(mid, w2_ref[...], preferred_element_type=jnp.float32)
            + b2_ref[...]
        )
        out_ref[...] = _ln_body(h, g_ref[...], b_ref[...])

    vec = lambda i: (0, 0)
    return pl.pallas_call(
        body,
        grid=(m // bm,),
        in_specs=[
            pl.BlockSpec((bm, _D), lambda i: (i, 0)),
            pl.BlockSpec((_D, _FF), vec),
            pl.BlockSpec((1, _FF), vec),
            pl.BlockSpec((_FF, _D), vec),
            pl.BlockSpec((1, _D), vec),
            pl.BlockSpec((1, _D), vec),
            pl.BlockSpec((1, _D), vec),
        ],
        out_specs=pl.BlockSpec((bm, _D), lambda i: (i, 0)),
        out_shape=jax.ShapeDtypeStruct((m, _D), jnp.float32),
    )(x, w1, b1.reshape(1, _FF), w2, b2.reshape(1, _D), g.reshape(1, _D),
      b.reshape(1, _D))


def _encoder_layer(x, wqkv, bqkv, wo, bo, w1, b1, w2, b2, g1, be1, g2, be2):
    qkv = _mm_bias(x, wqkv, bqkv)
    o = _attention(qkv, qkv, 0, _D // 128, 2 * _D // 128)
    x = _proj_ln(x, o, wo, bo, g1, be1)
    return _ffn_ln(x, w1, b1, w2, b2, g2, be2)


def _route(lc_last, gate_w, gate_b):
    scale = math.sqrt(_D)

    def body(x_ref, w_ref, b_ref, jw_ref, idx_ref):
        logits = scale * lax.dot_general(
            x_ref[...], w_ref[...], (((1,), (1,)), ((), ())),
            preferred_element_type=jnp.float32,
        ) + b_ref[...]
        iota = lax.broadcasted_iota(jnp.int32, (1, _P), 1)
        vals = logits
        tv, ti = [], []
        for _ in range(_K):
            mv = jnp.max(vals)
            ix = jnp.min(jnp.where(vals == mv, iota, _P))
            tv.append(mv)
            ti.append(ix)
            vals = jnp.where(iota == ix, -1e30, vals)
        m0 = tv[0]
        evec = jnp.exp(logits - m0)
        tot = jnp.float32(0.0)
        num = []
        for i in range(_K):
            ei = jnp.sum(jnp.where(iota == ti[i], evec, 0.0))
            num.append(ei)
            tot = tot + ei
        for i in range(_K):
            jw_ref[i] = num[i] / tot
            idx_ref[i] = ti[i]

    return pl.pallas_call(
        body,
        in_specs=[
            pl.BlockSpec((1, _D), lambda: (0, 0)),
            pl.BlockSpec((_P, _D), lambda: (0, 0)),
            pl.BlockSpec((1, _P), lambda: (0, 0)),
        ],
        out_specs=[
            pl.BlockSpec(memory_space=pltpu.SMEM),
            pl.BlockSpec(memory_space=pltpu.SMEM),
        ],
        out_shape=[
            jax.ShapeDtypeStruct((_K,), jnp.float32),
            jax.ShapeDtypeStruct((_K,), jnp.int32),
        ],
    )(lc_last, gate_w, gate_b.reshape(1, _P))


def _combine(jw, resp):
    bm = 256

    def body(jw_ref, r_ref, o_ref):
        acc = jw_ref[0] * r_ref[0]
        for kk in range(1, _K):
            acc = acc + jw_ref[kk] * r_ref[kk]
        o_ref[...] = acc

    return pl.pallas_call(
        body,
        grid=(_S // bm,),
        in_specs=[
            pl.BlockSpec(memory_space=pltpu.SMEM),
            pl.BlockSpec((_K, bm, _D), lambda i: (0, i, 0)),
        ],
        out_specs=pl.BlockSpec((bm, _D), lambda i: (i, 0)),
        out_shape=jax.ShapeDtypeStruct((_S, _D), jnp.float32),
    )(jw, resp)


_BV = 1024
_NV = (_V + _BV - 1) // _BV


def _decode_lse(h, dec_w, lab):
    def body(h_ref, w_ref, lab_ref, out_ref, ll_ref, m_s, s_s, la_s):
        v = pl.program_id(0)

        @pl.when(v == 0)
        def _():
            m_s[...] = jnp.full_like(m_s, -1e30)
            s_s[...] = jnp.zeros_like(s_s)
            la_s[...] = jnp.zeros_like(la_s)

        logits = lax.dot_general(
            h_ref[...], w_ref[...], (((1,), (1,)), ((), ())),
            preferred_element_type=jnp.float32,
        )
        out_ref[...] = logits
        col = lax.broadcasted_iota(jnp.int32, (_S, _BV), 1) + v * _BV
        lm = jnp.where(col < _V, logits, -1e30)
        lab_col = lab_ref[...]
        la_s[...] = la_s[...] + jnp.sum(
            jnp.where(col == lab_col, logits, 0.0), axis=1, keepdims=True)
        bmax = jnp.max(lm, axis=1, keepdims=True)
        esum = jnp.sum(jnp.exp(lm - bmax), axis=1, keepdims=True)
        m_old = m_s[...]
        m_new = jnp.maximum(m_old, bmax)
        s_new = s_s[...] * jnp.exp(m_old - m_new) + jnp.exp(bmax - m_new) * esum
        m_s[...] = m_new
        s_s[...] = s_new

        @pl.when(v == _NV - 1)
        def _():
            ll_ref[...] = la_s[...] - (m_new + jnp.log(s_new))

    return pl.pallas_call(
        body,
        grid=(_NV,),
        in_specs=[
            pl.BlockSpec((_S, _D), lambda v: (0, 0)),
            pl.BlockSpec((_BV, _D), lambda v: (v, 0)),
            pl.BlockSpec((_S, 1), lambda v: (0, 0)),
        ],
        out_specs=[
            pl.BlockSpec((_S, _BV), lambda v: (0, v)),
            pl.BlockSpec((_S, 8), lambda v: (0, 0)),
        ],
        out_shape=[
            jax.ShapeDtypeStruct((_S, _V), jnp.float32),
            jax.ShapeDtypeStruct((_S, 8), jnp.float32),
        ],
        scratch_shapes=[
            pltpu.VMEM((_S, 8), jnp.float32),
            pltpu.VMEM((_S, 8), jnp.float32),
            pltpu.VMEM((_S, 8), jnp.float32),
        ],
    )(h, dec_w, lab.reshape(_S, 1))


def _finalize(ll, tw):
    def body(ll_ref, tw_ref, out_ref):
        llv = jnp.max(ll_ref[...], axis=1, keepdims=True)
        row = lax.broadcasted_iota(jnp.int32, (_S, 1), 0)
        ce = -jnp.sum(jnp.where(row < _S - 1, llv, 0.0)) / (_S - 1)
        twv = tw_ref[...]
        mu = jnp.sum(twv) / _P
        var = jnp.sum((twv - mu) * (twv - mu)) / (_P - 1)
        out_ref[0, 0] = ce + _IMPORTANCE * var / (mu * mu)

    return pl.pallas_call(
        body,
        in_specs=[
            pl.BlockSpec((_S, 8), lambda: (0, 0)),
            pl.BlockSpec((1, _P), lambda: (0, 0)),
        ],
        out_specs=pl.BlockSpec(memory_space=pltpu.SMEM),
        out_shape=jax.ShapeDtypeStruct((1, 1), jnp.float32),
    )(ll, tw)


def kernel(inputs, responses, emb_table, dec_w, gate_w, gate_b, total_weights,
           le_wqkv, le_bqkv, le_wo, le_bo, le_w1, le_b1, le_w2, le_b2,
           le_ln1g, le_ln1b, le_ln2g, le_ln2b,
           enc_wqkv, enc_bqkv, enc_wo, enc_bo, enc_w1, enc_b1, enc_w2, enc_b2,
           enc_ln1g, enc_ln1b, enc_ln2g, enc_ln2b):
    idx = inputs[0].astype(jnp.int32)

    emb = _sc_gather(emb_table, idx)
    kv = _mm_bias(emb, le_wqkv[:, _D:], le_bqkv[_D:])
    q_last = _mm_bias(emb[_S - 1:], le_wqkv[:, :_D], le_bqkv[:_D])
    o = _attention(q_last, kv, 0, 0, _D // 128)
    x1 = _proj_ln(emb[_S - 1:], o, le_wo, le_bo, le_ln1g, le_ln1b)
    lc_last = _ffn_ln(x1, le_w1, le_b1, le_w2, le_b2, le_ln2g, le_ln2b)

    jw, topk_idx = _route(lc_last, gate_w, gate_b)
    h = _combine(jw, responses.reshape(_K, _S, _D))

    for l in range(_L):
        h = _encoder_layer(
            h, enc_wqkv[l], enc_bqkv[l], enc_wo[l], enc_bo[l],
            enc_w1[l], enc_b1[l], enc_w2[l], enc_b2[l],
            enc_ln1g[l], enc_ln1b[l], enc_ln2g[l], enc_ln2b[l])

    lab = jnp.concatenate([idx[1:], jnp.zeros((1,), jnp.int32)])
    decoded, ll = _decode_lse(h, dec_w, lab)
    loss = _finalize(ll, total_weights.reshape(1, _P))

    return (loss.reshape(()), decoded.reshape(1, _S, _V), topk_idx)


# speedup vs baseline: 1.6731x; 1.0225x over previous
import functools
import math

import jax
import jax.numpy as jnp
from jax import lax
from jax.experimental import pallas as pl
from jax.experimental.pallas import tpu as pltpu
from jax.experimental.pallas import tpu_sc as plsc

_B, _S, _D, _H, _DH, _FF, _V, _P, _K, _L = 1, 2048, 1024, 16, 64, 4096, 50257, 64, 8, 2
_IMPORTANCE = 3.0


def _sc_gather(table, idx):
    info = plsc.get_sparse_core_info()
    nc, ns = info.num_cores, info.num_subcores
    nw = nc * ns
    n = idx.shape[0]
    d = table.shape[1]
    b_per_w = n // nw
    mesh = plsc.VectorSubcoreMesh(core_axis_name="c", subcore_axis_name="s")

    @functools.partial(
        pl.kernel,
        mesh=mesh,
        out_type=jax.ShapeDtypeStruct((n, d), table.dtype),
        scratch_types=[
            pltpu.VMEM((b_per_w,), jnp.int32),
            pltpu.VMEM((b_per_w, d), table.dtype),
            pltpu.SemaphoreType.DMA,
        ],
    )
    def k(table_hbm, idx_hbm, out_hbm, idx_v, rows_v, sem):
        wid = lax.axis_index("s") * nc + lax.axis_index("c")
        base = wid * b_per_w
        pltpu.sync_copy(idx_hbm.at[pl.ds(base, b_per_w)], idx_v)
        pltpu.async_copy(table_hbm.at[idx_v], rows_v, sem).wait()
        pltpu.sync_copy(rows_v, out_hbm.at[pl.ds(base, b_per_w)])

    return k(table, idx)


def _mm_bias(x, w, b, n0=0, nn=None, bm=512, bn=512):
    m, kd = x.shape
    if nn is None:
        nn = w.shape[1] - n0
    bm = min(bm, m)
    bn = min(bn, nn)
    while m % bm:
        bm //= 2
    while nn % bn or n0 % bn:
        bn //= 2
    j0 = n0 // bn

    def body(x_ref, w_ref, b_ref, o_ref):
        o_ref[...] = (
            jnp.dot(x_ref[...].astype(jnp.bfloat16),
                    w_ref[...].astype(jnp.bfloat16),
                    preferred_element_type=jnp.float32)
            + b_ref[...]
        )

    return pl.pallas_call(
        body,
        grid=(m // bm, nn // bn),
        in_specs=[
            pl.BlockSpec((bm, kd), lambda i, j: (i, 0)),
            pl.BlockSpec((kd, bn), lambda i, j: (0, j0 + j)),
            pl.BlockSpec((1, bn), lambda i, j: (0, j0 + j)),
        ],
        out_specs=pl.BlockSpec((bm, bn), lambda i, j: (i, j)),
        out_shape=jax.ShapeDtypeStruct((m, nn), jnp.float32),
    )(x, w, b.reshape(1, -1))


def _attention(qarr, kvarr, q_off, k_off, v_off):
    sq = qarr.shape[0]
    s = kvarr.shape[0]
    bq = min(512, sq)
    scale = 1.0 / math.sqrt(_DH)

    def body(q_ref, k_ref, v_ref, o_ref):
        q = q_ref[...]
        k = k_ref[...]
        v = v_ref[...]
        outs = []
        for t in (0, 1):
            qt = q[:, t * _DH:(t + 1) * _DH].astype(jnp.bfloat16)
            kt = k[:, t * _DH:(t + 1) * _DH].astype(jnp.bfloat16)
            vt = v[:, t * _DH:(t + 1) * _DH].astype(jnp.bfloat16)
            sc = lax.dot_general(
                qt, kt, (((1,), (1,)), ((), ())),
                preferred_element_type=jnp.float32) * scale
            mx = jnp.max(sc, axis=1, keepdims=True)
            e = jnp.exp(sc - mx)
            p = e / jnp.sum(e, axis=1, keepdims=True)
            outs.append(jnp.dot(p.astype(jnp.bfloat16), vt,
                                preferred_element_type=jnp.float32))
        o_ref[...] = jnp.concatenate(outs, axis=1)

    return pl.pallas_call(
        body,
        grid=(_H // 2, sq // bq),
        in_specs=[
            pl.BlockSpec((bq, 128), lambda h, i: (i, q_off + h)),
            pl.BlockSpec((s, 128), lambda h, i: (0, k_off + h)),
            pl.BlockSpec((s, 128), lambda h, i: (0, v_off + h)),
        ],
        out_specs=pl.BlockSpec((bq, 128), lambda h, i: (i, h)),
        out_shape=jax.ShapeDtypeStruct((sq, _D), jnp.float32),
    )(qarr, kvarr, kvarr)


def _ln_body(h, g, b):
    mu = jnp.mean(h, axis=1, keepdims=True)
    var = jnp.mean((h - mu) * (h - mu), axis=1, keepdims=True)
    return (h - mu) * lax.rsqrt(var + 1e-5) * g + b


def _proj_ffn_ln(x, o, wo, bo, g1, be1, w1, b1, w2, b2, g2, be2):
    m = x.shape[0]
    bm = min(256, m)

    def body(x_ref, o_ref, wo_ref, bo_ref, g1_ref, be1_ref,
             w1_ref, b1_ref, w2_ref, b2_ref, g2_ref, be2_ref, out_ref):
        bf = jnp.bfloat16
        h = (
            x_ref[...]
            + jnp.dot(o_ref[...].astype(bf), wo_ref[...].astype(bf),
                      preferred_element_type=jnp.float32)
            + bo_ref[...]
        )
        h = _ln_body(h, g1_ref[...], be1_ref[...])
        mid = jnp.maximum(
            jnp.dot(h.astype(bf), w1_ref[...].astype(bf),
                    preferred_element_type=jnp.float32) + b1_ref[...],
            0.0,
        )
        h2 = (
            h
            + jnp.dot(mid.astype(bf), w2_ref[...].astype(bf),
                      preferred_element_type=jnp.float32)
            + b2_ref[...]
        )
        out_ref[...] = _ln_body(h2, g2_ref[...], be2_ref[...])

    vec = lambda i: (0, 0)
    return pl.pallas_call(
        body,
        grid=(m // bm,),
        in_specs=[
            pl.BlockSpec((bm, _D), lambda i: (i, 0)),
            pl.BlockSpec((bm, _D), lambda i: (i, 0)),
            pl.BlockSpec((_D, _D), vec),
            pl.BlockSpec((1, _D), vec),
            pl.BlockSpec((1, _D), vec),
            pl.BlockSpec((1, _D), vec),
            pl.BlockSpec((_D, _FF), vec),
            pl.BlockSpec((1, _FF), vec),
            pl.BlockSpec((_FF, _D), vec),
            pl.BlockSpec((1, _D), vec),
            pl.BlockSpec((1, _D), vec),
            pl.BlockSpec((1, _D), vec),
        ],
        out_specs=pl.BlockSpec((bm, _D), lambda i: (i, 0)),
        out_shape=jax.ShapeDtypeStruct((m, _D), jnp.float32),
    )(x, o, wo, bo.reshape(1, _D), g1.reshape(1, _D), be1.reshape(1, _D),
      w1, b1.reshape(1, _FF), w2, b2.reshape(1, _D), g2.reshape(1, _D),
      be2.reshape(1, _D))


def _encoder_layer(x, wqkv, bqkv, wo, bo, w1, b1, w2, b2, g1, be1, g2, be2):
    qkv = _mm_bias(x, wqkv, bqkv)
    o = _attention(qkv, qkv, 0, _D // 128, 2 * _D // 128)
    return _proj_ffn_ln(x, o, wo, bo, g1, be1, w1, b1, w2, b2, g2, be2)


def _route(lc_last, gate_w, gate_b):
    scale = math.sqrt(_D)

    def body(x_ref, w_ref, b_ref, jw_ref, idx_ref):
        logits = scale * lax.dot_general(
            x_ref[...], w_ref[...], (((1,), (1,)), ((), ())),
            preferred_element_type=jnp.float32,
        ) + b_ref[...]
        iota = lax.broadcasted_iota(jnp.int32, (1, _P), 1)
        vals = logits
        tv, ti = [], []
        for _ in range(_K):
            mv = jnp.max(vals)
            ix = jnp.min(jnp.where(vals == mv, iota, _P))
            tv.append(mv)
            ti.append(ix)
            vals = jnp.where(iota == ix, -1e30, vals)
        m0 = tv[0]
        evec = jnp.exp(logits - m0)
        tot = jnp.float32(0.0)
        num = []
        for i in range(_K):
            ei = jnp.sum(jnp.where(iota == ti[i], evec, 0.0))
            num.append(ei)
            tot = tot + ei
        for i in range(_K):
            jw_ref[i] = num[i] / tot
            idx_ref[i] = ti[i]

    return pl.pallas_call(
        body,
        in_specs=[
            pl.BlockSpec((1, _D), lambda: (0, 0)),
            pl.BlockSpec((_P, _D), lambda: (0, 0)),
            pl.BlockSpec((1, _P), lambda: (0, 0)),
        ],
        out_specs=[
            pl.BlockSpec(memory_space=pltpu.SMEM),
            pl.BlockSpec(memory_space=pltpu.SMEM),
        ],
        out_shape=[
            jax.ShapeDtypeStruct((_K,), jnp.float32),
            jax.ShapeDtypeStruct((_K,), jnp.int32),
        ],
    )(lc_last, gate_w, gate_b.reshape(1, _P))


def _combine(jw, resp):
    bm = 256

    def body(jw_ref, r_ref, o_ref):
        acc = jw_ref[0] * r_ref[0]
        for kk in range(1, _K):
            acc = acc + jw_ref[kk] * r_ref[kk]
        o_ref[...] = acc

    return pl.pallas_call(
        body,
        grid=(_S // bm,),
        in_specs=[
            pl.BlockSpec(memory_space=pltpu.SMEM),
            pl.BlockSpec((_K, bm, _D), lambda i: (0, i, 0)),
        ],
        out_specs=pl.BlockSpec((bm, _D), lambda i: (i, 0)),
        out_shape=jax.ShapeDtypeStruct((_S, _D), jnp.float32),
    )(jw, resp)


_BV = 1024
_NV = (_V + _BV - 1) // _BV


def _decode_lse(h, dec_w, lab):
    def body(h_ref, w_ref, lab_ref, out_ref, ll_ref, m_s, s_s, la_s):
        v = pl.program_id(0)

        @pl.when(v == 0)
        def _():
            m_s[...] = jnp.full_like(m_s, -1e30)
            s_s[...] = jnp.zeros_like(s_s)
            la_s[...] = jnp.zeros_like(la_s)

        logits = lax.dot_general(
            h_ref[...], w_ref[...], (((1,), (1,)), ((), ())),
            preferred_element_type=jnp.float32,
        )
        out_ref[...] = logits
        col = lax.broadcasted_iota(jnp.int32, (_S, _BV), 1) + v * _BV
        lm = jnp.where(col < _V, logits, -1e30)
        lab_col = lab_ref[...]
        la_s[...] = la_s[...] + jnp.sum(
            jnp.where(col == lab_col, logits, 0.0), axis=1, keepdims=True)
        bmax = jnp.max(lm, axis=1, keepdims=True)
        esum = jnp.sum(jnp.exp(lm - bmax), axis=1, keepdims=True)
        m_old = m_s[...]
        m_new = jnp.maximum(m_old, bmax)
        s_new = s_s[...] * jnp.exp(m_old - m_new) + jnp.exp(bmax - m_new) * esum
        m_s[...] = m_new
        s_s[...] = s_new

        @pl.when(v == _NV - 1)
        def _():
            ll_ref[...] = la_s[...] - (m_new + jnp.log(s_new))

    return pl.pallas_call(
        body,
        grid=(_NV,),
        in_specs=[
            pl.BlockSpec((_S, _D), lambda v: (0, 0)),
            pl.BlockSpec((_BV, _D), lambda v: (v, 0)),
            pl.BlockSpec((_S, 1), lambda v: (0, 0)),
        ],
        out_specs=[
            pl.BlockSpec((_S, _BV), lambda v: (0, v)),
            pl.BlockSpec((_S, 8), lambda v: (0, 0)),
        ],
        out_shape=[
            jax.ShapeDtypeStruct((_S, _V), jnp.float32),
            jax.ShapeDtypeStruct((_S, 8), jnp.float32),
        ],
        scratch_shapes=[
            pltpu.VMEM((_S, 8), jnp.float32),
            pltpu.VMEM((_S, 8), jnp.float32),
            pltpu.VMEM((_S, 8), jnp.float32),
        ],
    )(h, dec_w, lab.reshape(_S, 1))


def _finalize(ll, tw):
    def body(ll_ref, tw_ref, out_ref):
        llv = jnp.max(ll_ref[...], axis=1, keepdims=True)
        row = lax.broadcasted_iota(jnp.int32, (_S, 1), 0)
        ce = -jnp.sum(jnp.where(row < _S - 1, llv, 0.0)) / (_S - 1)
        twv = tw_ref[...]
        mu = jnp.sum(twv) / _P
        var = jnp.sum((twv - mu) * (twv - mu)) / (_P - 1)
        out_ref[0, 0] = ce + _IMPORTANCE * var / (mu * mu)

    return pl.pallas_call(
        body,
        in_specs=[
            pl.BlockSpec((_S, 8), lambda: (0, 0)),
            pl.BlockSpec((1, _P), lambda: (0, 0)),
        ],
        out_specs=pl.BlockSpec(memory_space=pltpu.SMEM),
        out_shape=jax.ShapeDtypeStruct((1, 1), jnp.float32),
    )(ll, tw)


def kernel(inputs, responses, emb_table, dec_w, gate_w, gate_b, total_weights,
           le_wqkv, le_bqkv, le_wo, le_bo, le_w1, le_b1, le_w2, le_b2,
           le_ln1g, le_ln1b, le_ln2g, le_ln2b,
           enc_wqkv, enc_bqkv, enc_wo, enc_bo, enc_w1, enc_b1, enc_w2, enc_b2,
           enc_ln1g, enc_ln1b, enc_ln2g, enc_ln2b):
    idx = inputs[0].astype(jnp.int32)

    emb = _sc_gather(emb_table, idx)
    kv = _mm_bias(emb, le_wqkv, le_bqkv, n0=_D)
    q_last = _mm_bias(emb[_S - 1:], le_wqkv, le_bqkv, n0=0, nn=_D)
    o = _attention(q_last, kv, 0, 0, _D // 128)
    lc_last = _proj_ffn_ln(emb[_S - 1:], o, le_wo, le_bo, le_ln1g, le_ln1b,
                           le_w1, le_b1, le_w2, le_b2, le_ln2g, le_ln2b)

    jw, topk_idx = _route(lc_last, gate_w, gate_b)
    h = _combine(jw, responses.reshape(_K, _S, _D))

    for l in range(_L):
        h = _encoder_layer(
            h, enc_wqkv[l], enc_bqkv[l], enc_wo[l], enc_bo[l],
            enc_w1[l], enc_b1[l], enc_w2[l], enc_b2[l],
            enc_ln1g[l], enc_ln1b[l], enc_ln2g[l], enc_ln2b[l])

    lab = jnp.concatenate([idx[1:], jnp.zeros((1,), jnp.int32)])
    decoded, ll = _decode_lse(h, dec_w, lab)
    loss = _finalize(ll, total_weights.reshape(1, _P))

    return (loss.reshape(()), decoded.reshape(1, _S, _V), topk_idx)


# speedup vs baseline: 1.6734x; 1.0002x over previous
import functools
import math

import jax
import jax.numpy as jnp
from jax import lax
from jax.experimental import pallas as pl
from jax.experimental.pallas import tpu as pltpu
from jax.experimental.pallas import tpu_sc as plsc

_B, _S, _D, _H, _DH, _FF, _V, _P, _K, _L = 1, 2048, 1024, 16, 64, 4096, 50257, 64, 8, 2
_IMPORTANCE = 3.0


def _sc_gather(table, idx):
    info = plsc.get_sparse_core_info()
    nc, ns = info.num_cores, info.num_subcores
    nw = nc * ns
    n = idx.shape[0]
    d = table.shape[1]
    b_per_w = n // nw
    mesh = plsc.VectorSubcoreMesh(core_axis_name="c", subcore_axis_name="s")

    @functools.partial(
        pl.kernel,
        mesh=mesh,
        out_type=jax.ShapeDtypeStruct((n, d), table.dtype),
        scratch_types=[
            pltpu.VMEM((b_per_w,), jnp.int32),
            pltpu.VMEM((b_per_w, d), table.dtype),
            pltpu.SemaphoreType.DMA,
        ],
    )
    def k(table_hbm, idx_hbm, out_hbm, idx_v, rows_v, sem):
        wid = lax.axis_index("s") * nc + lax.axis_index("c")
        base = wid * b_per_w
        pltpu.sync_copy(idx_hbm.at[pl.ds(base, b_per_w)], idx_v)
        pltpu.async_copy(table_hbm.at[idx_v], rows_v, sem).wait()
        pltpu.sync_copy(rows_v, out_hbm.at[pl.ds(base, b_per_w)])

    return k(table, idx)


def _mm_bias(x, w, b, n0=0, nn=None, bm=512, bn=512):
    m, kd = x.shape
    if nn is None:
        nn = w.shape[1] - n0
    bm = min(bm, m)
    bn = min(bn, nn)
    while m % bm:
        bm //= 2
    while nn % bn or n0 % bn:
        bn //= 2
    j0 = n0 // bn

    def body(x_ref, w_ref, b_ref, o_ref):
        o_ref[...] = (
            jnp.dot(x_ref[...], w_ref[...], preferred_element_type=jnp.float32)
            + b_ref[...]
        )

    return pl.pallas_call(
        body,
        grid=(m // bm, nn // bn),
        in_specs=[
            pl.BlockSpec((bm, kd), lambda i, j: (i, 0)),
            pl.BlockSpec((kd, bn), lambda i, j: (0, j0 + j)),
            pl.BlockSpec((1, bn), lambda i, j: (0, j0 + j)),
        ],
        out_specs=pl.BlockSpec((bm, bn), lambda i, j: (i, j)),
        out_shape=jax.ShapeDtypeStruct((m, nn), jnp.float32),
    )(x, w, b.reshape(1, -1))


def _attention(qarr, kvarr, q_off, k_off, v_off):
    sq = qarr.shape[0]
    s = kvarr.shape[0]
    bq = min(512, sq)
    scale = 1.0 / math.sqrt(_DH)

    def body(q_ref, k_ref, v_ref, o_ref):
        q = q_ref[...]
        k = k_ref[...]
        v = v_ref[...]
        outs = []
        for t in (0, 1):
            qt = q[:, t * _DH:(t + 1) * _DH]
            kt = k[:, t * _DH:(t + 1) * _DH]
            vt = v[:, t * _DH:(t + 1) * _DH]
            sc = lax.dot_general(
                qt, kt, (((1,), (1,)), ((), ())),
                preferred_element_type=jnp.float32) * scale
            mx = jnp.max(sc, axis=1, keepdims=True)
            e = jnp.exp(sc - mx)
            p = e / jnp.sum(e, axis=1, keepdims=True)
            outs.append(jnp.dot(p, vt, preferred_element_type=jnp.float32))
        o_ref[...] = jnp.concatenate(outs, axis=1)

    return pl.pallas_call(
        body,
        grid=(_H // 2, sq // bq),
        in_specs=[
            pl.BlockSpec((bq, 128), lambda h, i: (i, q_off + h)),
            pl.BlockSpec((s, 128), lambda h, i: (0, k_off + h)),
            pl.BlockSpec((s, 128), lambda h, i: (0, v_off + h)),
        ],
        out_specs=pl.BlockSpec((bq, 128), lambda h, i: (i, h)),
        out_shape=jax.ShapeDtypeStruct((sq, _D), jnp.float32),
    )(qarr, kvarr, kvarr)


def _ln_body(h, g, b):
    mu = jnp.mean(h, axis=1, keepdims=True)
    var = jnp.mean((h - mu) * (h - mu), axis=1, keepdims=True)
    return (h - mu) * lax.rsqrt(var + 1e-5) * g + b


def _proj_ffn_ln(x, o, wo, bo, g1, be1, w1, b1, w2, b2, g2, be2):
    m = x.shape[0]
    bm = min(256, m)

    def body(x_ref, o_ref, wo_ref, bo_ref, g1_ref, be1_ref,
             w1_ref, b1_ref, w2_ref, b2_ref, g2_ref, be2_ref, out_ref):
        h = (
            x_ref[...]
            + jnp.dot(o_ref[...], wo_ref[...],
                      preferred_element_type=jnp.float32)
            + bo_ref[...]
        )
        h = _ln_body(h, g1_ref[...], be1_ref[...])
        mid = jnp.maximum(
            jnp.dot(h, w1_ref[...],
                    preferred_element_type=jnp.float32) + b1_ref[...],
            0.0,
        )
        h2 = (
            h
            + jnp.dot(mid, w2_ref[...],
                      preferred_element_type=jnp.float32)
            + b2_ref[...]
        )
        out_ref[...] = _ln_body(h2, g2_ref[...], be2_ref[...])

    vec = lambda i: (0, 0)
    return pl.pallas_call(
        body,
        grid=(m // bm,),
        in_specs=[
            pl.BlockSpec((bm, _D), lambda i: (i, 0)),
            pl.BlockSpec((bm, _D), lambda i: (i, 0)),
            pl.BlockSpec((_D, _D), vec),
            pl.BlockSpec((1, _D), vec),
            pl.BlockSpec((1, _D), vec),
            pl.BlockSpec((1, _D), vec),
            pl.BlockSpec((_D, _FF), vec),
            pl.BlockSpec((1, _FF), vec),
            pl.BlockSpec((_FF, _D), vec),
            pl.BlockSpec((1, _D), vec),
            pl.BlockSpec((1, _D), vec),
            pl.BlockSpec((1, _D), vec),
        ],
        out_specs=pl.BlockSpec((bm, _D), lambda i: (i, 0)),
        out_shape=jax.ShapeDtypeStruct((m, _D), jnp.float32),
    )(x, o, wo, bo.reshape(1, _D), g1.reshape(1, _D), be1.reshape(1, _D),
      w1, b1.reshape(1, _FF), w2, b2.reshape(1, _D), g2.reshape(1, _D),
      be2.reshape(1, _D))


def _encoder_layer(x, wqkv, bqkv, wo, bo, w1, b1, w2, b2, g1, be1, g2, be2):
    qkv = _mm_bias(x, wqkv, bqkv)
    o = _attention(qkv, qkv, 0, _D // 128, 2 * _D // 128)
    return _proj_ffn_ln(x, o, wo, bo, g1, be1, w1, b1, w2, b2, g2, be2)


def _route(lc_last, gate_w, gate_b):
    scale = math.sqrt(_D)

    def body(x_ref, w_ref, b_ref, jw_ref, idx_ref):
        logits = scale * lax.dot_general(
            x_ref[...], w_ref[...], (((1,), (1,)), ((), ())),
            preferred_element_type=jnp.float32,
        ) + b_ref[...]
        iota = lax.broadcasted_iota(jnp.int32, (1, _P), 1)
        vals = logits
        tv, ti = [], []
        for _ in range(_K):
            mv = jnp.max(vals)
            ix = jnp.min(jnp.where(vals == mv, iota, _P))
            tv.append(mv)
            ti.append(ix)
            vals = jnp.where(iota == ix, -1e30, vals)
        m0 = tv[0]
        evec = jnp.exp(logits - m0)
        tot = jnp.float32(0.0)
        num = []
        for i in range(_K):
            ei = jnp.sum(jnp.where(iota == ti[i], evec, 0.0))
            num.append(ei)
            tot = tot + ei
        for i in range(_K):
            jw_ref[i] = num[i] / tot
            idx_ref[i] = ti[i]

    return pl.pallas_call(
        body,
        in_specs=[
            pl.BlockSpec((1, _D), lambda: (0, 0)),
            pl.BlockSpec((_P, _D), lambda: (0, 0)),
            pl.BlockSpec((1, _P), lambda: (0, 0)),
        ],
        out_specs=[
            pl.BlockSpec(memory_space=pltpu.SMEM),
            pl.BlockSpec(memory_space=pltpu.SMEM),
        ],
        out_shape=[
            jax.ShapeDtypeStruct((_K,), jnp.float32),
            jax.ShapeDtypeStruct((_K,), jnp.int32),
        ],
    )(lc_last, gate_w, gate_b.reshape(1, _P))


def _combine(jw, resp):
    bm = 256

    def body(jw_ref, r_ref, o_ref):
        acc = jw_ref[0] * r_ref[0]
        for kk in range(1, _K):
            acc = acc + jw_ref[kk] * r_ref[kk]
        o_ref[...] = acc

    return pl.pallas_call(
        body,
        grid=(_S // bm,),
        in_specs=[
            pl.BlockSpec(memory_space=pltpu.SMEM),
            pl.BlockSpec((_K, bm, _D), lambda i: (0, i, 0)),
        ],
        out_specs=pl.BlockSpec((bm, _D), lambda i: (i, 0)),
        out_shape=jax.ShapeDtypeStruct((_S, _D), jnp.float32),
    )(jw, resp)


_BV = 1024
_NV = (_V + _BV - 1) // _BV


def _decode_lse(h, dec_w, lab):
    def body(h_ref, w_ref, lab_ref, out_ref, ll_ref, m_s, s_s, la_s):
        v = pl.program_id(0)

        @pl.when(v == 0)
        def _():
            m_s[...] = jnp.full_like(m_s, -1e30)
            s_s[...] = jnp.zeros_like(s_s)
            la_s[...] = jnp.zeros_like(la_s)

        logits = lax.dot_general(
            h_ref[...], w_ref[...], (((1,), (1,)), ((), ())),
            preferred_element_type=jnp.float32,
        )
        out_ref[...] = logits
        col = lax.broadcasted_iota(jnp.int32, (_S, _BV), 1) + v * _BV
        lm = jnp.where(col < _V, logits, -1e30)
        lab_col = lab_ref[...]
        la_s[...] = la_s[...] + jnp.sum(
            jnp.where(col == lab_col, logits, 0.0), axis=1, keepdims=True)
        bmax = jnp.max(lm, axis=1, keepdims=True)
        esum = jnp.sum(jnp.exp(lm - bmax), axis=1, keepdims=True)
        m_old = m_s[...]
        m_new = jnp.maximum(m_old, bmax)
        s_new = s_s[...] * jnp.exp(m_old - m_new) + jnp.exp(bmax - m_new) * esum
        m_s[...] = m_new
        s_s[...] = s_new

        @pl.when(v == _NV - 1)
        def _():
            ll_ref[...] = la_s[...] - (m_new + jnp.log(s_new))

    return pl.pallas_call(
        body,
        grid=(_NV,),
        in_specs=[
            pl.BlockSpec((_S, _D), lambda v: (0, 0)),
            pl.BlockSpec((_BV, _D), lambda v: (v, 0)),
            pl.BlockSpec((_S, 1), lambda v: (0, 0)),
        ],
        out_specs=[
            pl.BlockSpec((_S, _BV), lambda v: (0, v)),
            pl.BlockSpec((_S, 8), lambda v: (0, 0)),
        ],
        out_shape=[
            jax.ShapeDtypeStruct((_S, _V), jnp.float32),
            jax.ShapeDtypeStruct((_S, 8), jnp.float32),
        ],
        scratch_shapes=[
            pltpu.VMEM((_S, 8), jnp.float32),
            pltpu.VMEM((_S, 8), jnp.float32),
            pltpu.VMEM((_S, 8), jnp.float32),
        ],
    )(h, dec_w, lab.reshape(_S, 1))


def _finalize(ll, tw):
    def body(ll_ref, tw_ref, out_ref):
        llv = jnp.max(ll_ref[...], axis=1, keepdims=True)
        row = lax.broadcasted_iota(jnp.int32, (_S, 1), 0)
        ce = -jnp.sum(jnp.where(row < _S - 1, llv, 0.0)) / (_S - 1)
        twv = tw_ref[...]
        mu = jnp.sum(twv) / _P
        var = jnp.sum((twv - mu) * (twv - mu)) / (_P - 1)
        out_ref[0, 0] = ce + _IMPORTANCE * var / (mu * mu)

    return pl.pallas_call(
        body,
        in_specs=[
            pl.BlockSpec((_S, 8), lambda: (0, 0)),
            pl.BlockSpec((1, _P), lambda: (0, 0)),
        ],
        out_specs=pl.BlockSpec(memory_space=pltpu.SMEM),
        out_shape=jax.ShapeDtypeStruct((1, 1), jnp.float32),
    )(ll, tw)


def kernel(inputs, responses, emb_table, dec_w, gate_w, gate_b, total_weights,
           le_wqkv, le_bqkv, le_wo, le_bo, le_w1, le_b1, le_w2, le_b2,
           le_ln1g, le_ln1b, le_ln2g, le_ln2b,
           enc_wqkv, enc_bqkv, enc_wo, enc_bo, enc_w1, enc_b1, enc_w2, enc_b2,
           enc_ln1g, enc_ln1b, enc_ln2g, enc_ln2b):
    idx = inputs[0].astype(jnp.int32)

    emb = _sc_gather(emb_table, idx)
    kv = _mm_bias(emb, le_wqkv, le_bqkv, n0=_D)
    q_last = _mm_bias(emb[_S - 1:], le_wqkv, le_bqkv, n0=0, nn=_D)
    o = _attention(q_last, kv, 0, 0, _D // 128)
    lc_last = _proj_ffn_ln(emb[_S - 1:], o, le_wo, le_bo, le_ln1g, le_ln1b,
                           le_w1, le_b1, le_w2, le_b2, le_ln2g, le_ln2b)

    jw, topk_idx = _route(lc_last, gate_w, gate_b)
    h = _combine(jw, responses.reshape(_K, _S, _D))

    for l in range(_L):
        h = _encoder_layer(
            h, enc_wqkv[l], enc_bqkv[l], enc_wo[l], enc_bo[l],
            enc_w1[l], enc_b1[l], enc_w2[l], enc_b2[l],
            enc_ln1g[l], enc_ln1b[l], enc_ln2g[l], enc_ln2b[l])

    lab = jnp.concatenate([idx[1:], jnp.zeros((1,), jnp.int32)])
    decoded, ll = _decode_lse(h, dec_w, lab)
    loss = _finalize(ll, total_weights.reshape(1, _P))

    return (loss.reshape(()), decoded.reshape(1, _S, _V), topk_idx)
